# baseline (speedup 1.0000x reference)
import jax
import jax.numpy as jnp
from jax.experimental import pallas as pl
from jax.experimental.pallas import tpu as pltpu

_B, _T, _A = 32, 2048, 64
_NS, _ED, _H, _D = 32, 64, 512, 1024
_G = 2


def _mlp_kernel(x_ref, at_ref, emb_ref, w1b_ref, b1_ref, w1a_ref, w2_ref,
                b2_ref, o_ref, c_ref):
    i = pl.program_id(0)

    @pl.when(i == 0)
    def _compute_c():
        at = at_ref[...]
        niota = jax.lax.broadcasted_iota(jnp.int32, (_NS, _B), 0)
        onehot_t = (niota == at).astype(jnp.float32)
        g = jax.lax.dot_general(onehot_t, emb_ref[...],
                                (((0,), (0,)), ((), ())),
                                preferred_element_type=jnp.float32)
        c = jnp.dot(g, w1b_ref[...],
                    preferred_element_type=jnp.float32) + b1_ref[...]
        c_ref[...] = c.reshape(_B // _G, _G, _H)

    x = x_ref[...].reshape(_G * _T, _A).astype(jnp.bfloat16)
    h = jnp.dot(x, w1a_ref[...], preferred_element_type=jnp.float32)
    cs = c_ref[i].astype(jnp.bfloat16)
    h = h.astype(jnp.bfloat16).reshape(_G, _T, _H) + cs[:, None, :]
    h = jnp.maximum(h, 0.0).reshape(_G * _T, _H)
    o = jnp.dot(h, w2_ref[...], preferred_element_type=jnp.float32) + b2_ref[...]
    o_ref[...] = o.reshape(_G, _T, _D)


def kernel(padded_action, action_type, emb, W1, b1, W2, b2):
    at2 = action_type.reshape(1, _B).astype(jnp.int32)
    w1a = W1[:_A].astype(jnp.bfloat16)
    w1b = W1[_A:]
    b1r = b1.reshape(1, _H)
    w2 = W2.astype(jnp.bfloat16)
    b2r = b2.reshape(1, _D)

    out = pl.pallas_call(
        _mlp_kernel,
        grid=(_B // _G,),
        out_shape=jax.ShapeDtypeStruct((_B, _T, _D), jnp.float32),
        in_specs=[
            pl.BlockSpec((_G, _T, _A), lambda i: (i, 0, 0)),
            pl.BlockSpec((1, _B), lambda i: (0, 0)),
            pl.BlockSpec((_NS, _ED), lambda i: (0, 0)),
            pl.BlockSpec((_ED, _H), lambda i: (0, 0)),
            pl.BlockSpec((1, _H), lambda i: (0, 0)),
            pl.BlockSpec((_A, _H), lambda i: (0, 0)),
            pl.BlockSpec((_H, _D), lambda i: (0, 0)),
            pl.BlockSpec((1, _D), lambda i: (0, 0)),
        ],
        out_specs=pl.BlockSpec((_G, _T, _D), lambda i: (i, 0, 0)),
        scratch_shapes=[pltpu.VMEM((_B // _G, _G, _H), jnp.float32)],
        compiler_params=pltpu.CompilerParams(
            dimension_semantics=("arbitrary",)),
    )(padded_action, at2, emb, w1b, b1r, w1a, w2, b2r)
    return out


# speedup vs baseline: 1.0011x; 1.0011x over previous
import jax
import jax.numpy as jnp
from jax.experimental import pallas as pl
from jax.experimental.pallas import tpu as pltpu

_B, _T, _A = 32, 2048, 64
_NS, _ED, _H, _D = 32, 64, 512, 1024
_G = 2


def _mlp_kernel(x_ref, at_ref, emb_ref, w1b_ref, b1_ref, w1a_ref, w2_ref,
                b2_ref, o_ref, c_ref):
    i = pl.program_id(0)

    @pl.when(i == 0)
    def _compute_c():
        at = at_ref[...]
        niota = jax.lax.broadcasted_iota(jnp.int32, (_NS, _B), 0)
        onehot_t = (niota == at).astype(jnp.float32)
        g = jax.lax.dot_general(onehot_t, emb_ref[...],
                                (((0,), (0,)), ((), ())),
                                preferred_element_type=jnp.float32)
        c = jnp.dot(g, w1b_ref[...],
                    preferred_element_type=jnp.float32) + b1_ref[...]
        c_ref[...] = c.reshape(_B // _G, _G, _H)

    x = x_ref[...].reshape(_G * _T, _A).astype(jnp.bfloat16)
    h = jnp.dot(x, w1a_ref[...], preferred_element_type=jnp.float32)
    cs = c_ref[i]
    h = h.reshape(_G, _T, _H) + cs[:, None, :]
    h = jnp.maximum(h, 0.0).reshape(_G * _T, _H).astype(jnp.bfloat16)
    o = jnp.dot(h, w2_ref[...], preferred_element_type=jnp.float32) + b2_ref[...]
    o_ref[...] = o.reshape(_G, _T, _D)


def kernel(padded_action, action_type, emb, W1, b1, W2, b2):
    at2 = action_type.reshape(1, _B).astype(jnp.int32)
    w1a = W1[:_A].astype(jnp.bfloat16)
    w1b = W1[_A:]
    b1r = b1.reshape(1, _H)
    w2 = W2.astype(jnp.bfloat16)
    b2r = b2.reshape(1, _D)

    out = pl.pallas_call(
        _mlp_kernel,
        grid=(_B // _G,),
        out_shape=jax.ShapeDtypeStruct((_B, _T, _D), jnp.float32),
        in_specs=[
            pl.BlockSpec((_G, _T, _A), lambda i: (i, 0, 0)),
            pl.BlockSpec((1, _B), lambda i: (0, 0)),
            pl.BlockSpec((_NS, _ED), lambda i: (0, 0)),
            pl.BlockSpec((_ED, _H), lambda i: (0, 0)),
            pl.BlockSpec((1, _H), lambda i: (0, 0)),
            pl.BlockSpec((_A, _H), lambda i: (0, 0)),
            pl.BlockSpec((_H, _D), lambda i: (0, 0)),
            pl.BlockSpec((1, _D), lambda i: (0, 0)),
        ],
        out_specs=pl.BlockSpec((_G, _T, _D), lambda i: (i, 0, 0)),
        scratch_shapes=[pltpu.VMEM((_B // _G, _G, _H), jnp.float32)],
        compiler_params=pltpu.CompilerParams(
            dimension_semantics=("arbitrary",)),
    )(padded_action, at2, emb, w1b, b1r, w1a, w2, b2r)
    return out
